# baseline (speedup 1.0000x reference)
import jax
import jax.numpy as jnp
from jax.experimental import pallas as pl
from jax.experimental.pallas import tpu as pltpu

MAX_LEN = 8192
DIM = 1024
NSLICE = 8
SLICE_ROWS = MAX_LEN // NSLICE
CHUNK = 256
NCHUNK = MAX_LEN // CHUNK


def _body(t_ref, pe_ref, zeros_ref, out_ref, sems, csem):
    t = t_ref[0]

    @pl.when(t >= MAX_LEN)
    def _fast():
        copies = []
        for k in range(NSLICE):
            sl = pl.ds(k * SLICE_ROWS, SLICE_ROWS)
            c = pltpu.make_async_copy(pe_ref.at[sl], out_ref.at[sl], sems.at[k])
            c.start()
            copies.append(c)
        for c in copies:
            c.wait()

    @pl.when(t < MAX_LEN)
    def _masked():
        for ci in range(NCHUNK):
            cbase = ci * CHUNK
            sl = pl.ds(cbase, CHUNK)

            @pl.when(cbase + CHUNK <= t)
            def _copy_chunk():
                c = pltpu.make_async_copy(pe_ref.at[sl], out_ref.at[sl], csem)
                c.start()
                c.wait()

            @pl.when(cbase >= t)
            def _zero_chunk():
                c = pltpu.make_async_copy(zeros_ref, out_ref.at[sl], csem)
                c.start()
                c.wait()

            @pl.when(jnp.logical_and(cbase < t, cbase + CHUNK > t))
            def _straddle():
                def row_body(r, carry):
                    row = cbase + r

                    @pl.when(row < t)
                    def _copy_row():
                        c = pltpu.make_async_copy(
                            pe_ref.at[row], out_ref.at[row], csem)
                        c.start()
                        c.wait()

                    @pl.when(row >= t)
                    def _zero_row():
                        c = pltpu.make_async_copy(
                            zeros_ref.at[0], out_ref.at[row], csem)
                        c.start()
                        c.wait()

                    return carry

                jax.lax.fori_loop(0, CHUNK, row_body, 0)


def kernel(pe, T):
    t_arr = jnp.asarray(T, dtype=jnp.int32).reshape((1,))
    zeros = jnp.zeros((CHUNK, DIM), dtype=jnp.float32)
    out = pl.pallas_call(
        _body,
        in_specs=[
            pl.BlockSpec(memory_space=pltpu.SMEM),
            pl.BlockSpec(memory_space=pl.ANY),
            pl.BlockSpec(memory_space=pl.ANY),
        ],
        out_specs=pl.BlockSpec(memory_space=pl.ANY),
        out_shape=jax.ShapeDtypeStruct((MAX_LEN, DIM), jnp.float32),
        scratch_shapes=[
            pltpu.SemaphoreType.DMA((NSLICE,)),
            pltpu.SemaphoreType.DMA,
        ],
    )(t_arr, pe, zeros)
    return out[None, :, :]


# speedup vs baseline: 38.9831x; 38.9831x over previous
import jax
import jax.numpy as jnp
from jax.experimental import pallas as pl
from jax.experimental.pallas import tpu as pltpu

MAX_LEN = 8192
DIM = 1024
CHUNK = 512
NCHUNK = MAX_LEN // CHUNK
NBUF = 4


def _body(t_ref, pe_ref, zeros_ref, out_ref, bufs, sin, sout, csem):
    t = t_ref[0]

    @pl.when(t >= MAX_LEN)
    def _fast():
        n = NCHUNK
        h_in = [None] * n
        h_out = [None] * n
        for i in range(n):
            b = i % NBUF
            if i >= NBUF:
                h_out[i - NBUF].wait()
            src = pe_ref.at[pl.ds(i * CHUNK, CHUNK)]
            h_in[i] = pltpu.make_async_copy(src, bufs.at[b], sin.at[b])
            h_in[i].start()
            if i >= 1:
                h_in[i - 1].wait()
                pb = (i - 1) % NBUF
                dst = out_ref.at[pl.ds((i - 1) * CHUNK, CHUNK)]
                h_out[i - 1] = pltpu.make_async_copy(bufs.at[pb], dst, sout.at[pb])
                h_out[i - 1].start()
        h_in[n - 1].wait()
        lb = (n - 1) % NBUF
        dst = out_ref.at[pl.ds((n - 1) * CHUNK, CHUNK)]
        h_out[n - 1] = pltpu.make_async_copy(bufs.at[lb], dst, sout.at[lb])
        h_out[n - 1].start()
        for i in range(n - NBUF, n):
            h_out[i].wait()

    @pl.when(t < MAX_LEN)
    def _masked():
        for ci in range(NCHUNK):
            cbase = ci * CHUNK
            sl = pl.ds(cbase, CHUNK)

            def _sync(src, dst):
                c = pltpu.make_async_copy(src, dst, csem)
                c.start()
                c.wait()

            @pl.when(cbase + CHUNK <= t)
            def _copy_chunk():
                _sync(pe_ref.at[sl], bufs.at[0])
                _sync(bufs.at[0], out_ref.at[sl])

            @pl.when(cbase >= t)
            def _zero_chunk():
                _sync(zeros_ref, bufs.at[0])
                _sync(bufs.at[0], out_ref.at[sl])

            @pl.when(jnp.logical_and(cbase < t, cbase + CHUNK > t))
            def _straddle():
                def row_body(r, carry):
                    row = cbase + r

                    @pl.when(row < t)
                    def _copy_row():
                        _sync(pe_ref.at[row], out_ref.at[row])

                    @pl.when(row >= t)
                    def _zero_row():
                        _sync(zeros_ref.at[0], out_ref.at[row])

                    return carry

                jax.lax.fori_loop(0, CHUNK, row_body, 0)


def kernel(pe, T):
    t_arr = jnp.asarray(T, dtype=jnp.int32).reshape((1,))
    zeros = jnp.zeros((CHUNK, DIM), dtype=jnp.float32)
    out = pl.pallas_call(
        _body,
        in_specs=[
            pl.BlockSpec(memory_space=pltpu.SMEM),
            pl.BlockSpec(memory_space=pl.ANY),
            pl.BlockSpec(memory_space=pl.ANY),
        ],
        out_specs=pl.BlockSpec(memory_space=pl.ANY),
        out_shape=jax.ShapeDtypeStruct((MAX_LEN, DIM), jnp.float32),
        scratch_shapes=[
            pltpu.VMEM((NBUF, CHUNK, DIM), jnp.float32),
            pltpu.SemaphoreType.DMA((NBUF,)),
            pltpu.SemaphoreType.DMA((NBUF,)),
            pltpu.SemaphoreType.DMA,
        ],
    )(t_arr, pe, zeros)
    return out[None, :, :]
